# Initial kernel scaffold; baseline (speedup 1.0000x reference)
import jax
import jax.numpy as jnp
from jax.experimental import pallas as pl


def kernel(x, edge_attr, edge_index, edge_weight, batch_index, params):
    raise NotImplementedError("write your pallas kernel here")


# speedup vs baseline: 4.4220x; 4.4220x over previous
import functools
import math

import jax
import jax.numpy as jnp
from jax.experimental import pallas as pl
from jax.experimental.pallas import tpu as pltpu

_N = 10000
_E = 160000
_H = 4
_C = 256
_HC = _H * _C
_DE = 16
_TE = 256
_NEG = -3.0e38

_KVX_W = 2320
_QQE_W = 1104
_M_W = 1408


def _mm_kernel(a_ref, w_ref, b_ref, o_ref, *, relu):
    acc = jnp.dot(a_ref[...], w_ref[...], preferred_element_type=jnp.float32)
    acc = acc + b_ref[...]
    if relu:
        acc = jnp.maximum(acc, 0.0)
    o_ref[...] = acc


def _mm(a, w_t, b=None, relu=False, mt=1000, nt=512):
    m, k = a.shape
    nout = w_t.shape[1]
    if b is None:
        b = jnp.zeros((1, nout), jnp.float32)
    else:
        b = b.reshape(1, nout)
    nt = min(nt, nout)
    grid = (m // mt, nout // nt)
    return pl.pallas_call(
        functools.partial(_mm_kernel, relu=relu),
        grid=grid,
        in_specs=[
            pl.BlockSpec((mt, k), lambda i, j: (i, 0)),
            pl.BlockSpec((k, nt), lambda i, j: (0, j)),
            pl.BlockSpec((1, nt), lambda i, j: (0, j)),
        ],
        out_specs=pl.BlockSpec((mt, nt), lambda i, j: (i, j)),
        out_shape=jax.ShapeDtypeStruct((m, nout), jnp.float32),
    )(a, w_t, b)


def _edge_kernel(kvx_ref, qqe_ref, ea_ref, ew_ref, tril_ref, o_ref, carry_ref):
    t = pl.program_id(0)
    nt = pl.num_programs(0)

    @pl.when(t == 0)
    def _():
        carry_ref[...] = jnp.zeros_like(carry_ref)

    @pl.when(t < nt - 1)
    def _():
        kvx = kvx_ref[...]
        qqe = qqe_ref[...]
        ea = ea_ref[...]
        pieces = []
        exs = []
        for h in range(_H):
            qh = qqe[:, h * _C:(h + 1) * _C]
            kh = kvx[:, h * _C:(h + 1) * _C]
            qeh = qqe[:, _HC + h * _DE:_HC + (h + 1) * _DE]
            dot = jnp.sum(qh * kh, axis=1, keepdims=True)
            dot = dot + jnp.sum(ea * qeh, axis=1, keepdims=True)
            ex = jnp.exp(dot * (1.0 / math.sqrt(_C)))
            exs.append(ex)
            vh = kvx[:, _HC + h * _C:_HC + (h + 1) * _C]
            pieces.append(ex * vh)
        norm = (kvx[:, 2 * _HC + _C:2 * _HC + _C + 1] * ew_ref[...]
                * qqe[:, _HC + _H * _DE:_HC + _H * _DE + 1])
        pieces.append(norm * kvx[:, 2 * _HC:2 * _HC + _C])
        for h in range(_H):
            pieces.append(exs[h] * ea)
        pieces.append(jnp.concatenate(exs, axis=1))
        pieces.append(jnp.zeros((_TE, _M_W - _HC - _C - _H * _DE - _H),
                                jnp.float32))
        m = jnp.concatenate(pieces, axis=1)
        o_ref[...] = (jnp.dot(tril_ref[...], m,
                              preferred_element_type=jnp.float32)
                      + carry_ref[...])
        carry_ref[...] = carry_ref[...] + jnp.sum(m, axis=0, keepdims=True)

    @pl.when(t == nt - 1)
    def _():
        o_ref[...] = jnp.broadcast_to(carry_ref[...], o_ref.shape)


def _edge_csum(kvx_g, qqe_g, ea_s, ew_s, tril):
    grid = (_E // _TE + 1,)
    return pl.pallas_call(
        _edge_kernel,
        grid=grid,
        in_specs=[
            pl.BlockSpec((_TE, _KVX_W), lambda t: (t, 0)),
            pl.BlockSpec((_TE, _QQE_W), lambda t: (t, 0)),
            pl.BlockSpec((_TE, _DE), lambda t: (t, 0)),
            pl.BlockSpec((_TE, 1), lambda t: (t, 0)),
            pl.BlockSpec((_TE, _TE), lambda t: (0, 0)),
        ],
        out_specs=pl.BlockSpec((_TE, _M_W), lambda t: (t, 0)),
        out_shape=jax.ShapeDtypeStruct((_E + _TE, _M_W), jnp.float32),
        scratch_shapes=[pltpu.VMEM((1, _M_W), jnp.float32)],
    )(kvx_g, qqe_g, ea_s, ew_s, tril)


def _bn_kernel(x_ref, g_ref, b_ref, o_ref, *, relu):
    x = x_ref[...]
    if relu:
        x = jnp.maximum(x, 0.0)
    mu = jnp.mean(x, axis=0, keepdims=True)
    var = jnp.mean(jnp.square(x - mu), axis=0, keepdims=True)
    o_ref[...] = (x - mu) * jax.lax.rsqrt(var + 1e-5) * g_ref[...] + b_ref[...]


def _bn(x, g, b, relu):
    n, d = x.shape
    return pl.pallas_call(
        functools.partial(_bn_kernel, relu=relu),
        grid=(1,),
        in_specs=[
            pl.BlockSpec((n, d), lambda i: (0, 0)),
            pl.BlockSpec((1, d), lambda i: (0, 0)),
            pl.BlockSpec((1, d), lambda i: (0, 0)),
        ],
        out_specs=pl.BlockSpec((n, d), lambda i: (0, 0)),
        out_shape=jax.ShapeDtypeStruct((n, d), jnp.float32),
    )(x, g.reshape(1, d), b.reshape(1, d))


def _pool_kernel(h_ref, w_ref, o_ref, *, kkeep):
    h = h_ref[...]
    w = w_ref[...]
    wnorm = jnp.sqrt(jnp.sum(w * w))
    score = jnp.tanh(jnp.sum(h * w, axis=1, keepdims=True) / wnorm)
    u = jax.lax.bitcast_convert_type(score, jnp.int32)
    m = u ^ ((u >> 31) & jnp.int32(0x7FFFFFFF))

    lo0 = jnp.int32(-1065353218)
    hi0 = jnp.int32(1065353217)

    def body(_, lohi):
        lo, hi = lohi
        mid = lo + ((hi - lo + 1) >> 1)
        cnt = jnp.sum(jnp.where(m >= mid, 1, 0))
        big = cnt >= kkeep
        return jnp.where(big, mid, lo), jnp.where(big, hi, mid - 1)

    lo, hi = jax.lax.fori_loop(0, 32, body, (lo0, hi0))
    tstar = lo
    c_gt = jnp.sum(jnp.where(m > tstar, 1, 0))
    need = kkeep - c_gt
    idx = jax.lax.broadcasted_iota(jnp.int32, m.shape, 0)
    eq = m == tstar

    def jbody(_, lohi):
        lo, hi = lohi
        mid = (lo + hi) >> 1
        cnt = jnp.sum(jnp.where(eq & (idx < mid), 1, 0))
        small = cnt < need
        return jnp.where(small, mid + 1, lo), jnp.where(small, hi, mid)

    jlo, _ = jax.lax.fori_loop(0, 15, jbody, (jnp.int32(0), jnp.int32(_N)))
    sel = (m > tstar) | (eq & (idx < jlo))
    hp = score * h
    gmax = jnp.max(jnp.where(sel, hp, _NEG), axis=0, keepdims=True)
    gmean = jnp.sum(jnp.where(sel, hp, 0.0), axis=0, keepdims=True) / kkeep
    o_ref[...] = jnp.concatenate([gmax, gmean], axis=1)


def _pool(h, w, kkeep):
    n, d = h.shape
    return pl.pallas_call(
        functools.partial(_pool_kernel, kkeep=kkeep),
        grid=(1,),
        in_specs=[
            pl.BlockSpec((n, d), lambda i: (0, 0)),
            pl.BlockSpec((1, d), lambda i: (0, 0)),
        ],
        out_specs=pl.BlockSpec((1, 2 * d), lambda i: (0, 0)),
        out_shape=jax.ShapeDtypeStruct((1, 2 * d), jnp.float32),
    )(h, w.reshape(1, d))


def _deg_kernel(row_ref, o_ref):
    t = pl.program_id(0)

    @pl.when(t == 0)
    def _():
        o_ref[...] = jnp.zeros_like(o_ref)

    row = row_ref[...]
    ids = jax.lax.broadcasted_iota(jnp.int32, (_TE, _N), 1)
    onehot = jnp.where(row == ids, 1.0, 0.0)
    o_ref[...] = o_ref[...] + jnp.sum(onehot, axis=0, keepdims=True)


def _deg(row):
    grid = (_E // _TE,)
    return pl.pallas_call(
        _deg_kernel,
        grid=grid,
        in_specs=[pl.BlockSpec((_TE, 1), lambda t: (t, 0))],
        out_specs=pl.BlockSpec((1, _N), lambda t: (0, 0)),
        out_shape=jax.ShapeDtypeStruct((1, _N), jnp.float32),
    )(row.reshape(_E, 1))


def _block_diag_we(w_e):
    we = w_e.reshape(_H, _C, _DE)
    bd = jnp.zeros((_HC, _H * _DE), jnp.float32)
    for h in range(_H):
        bd = bd.at[h * _C:(h + 1) * _C, h * _DE:(h + 1) * _DE].set(we[h])
    return bd


def _conv_layer(x, p, prep):
    (row_s, col_s, ea_s, ew_s, indptr, dis, tril, bn_g, bn_b, t_w, t_b) = prep
    w_cat = jnp.concatenate(
        [p["W_q"].T, p["W_k"].T, p["W_v"].T, p["W_skip"].T], axis=1)
    b_cat = jnp.concatenate([p["b_q"], p["b_k"], p["b_v"], p["b_skip"]])
    qkvs = _mm(x, w_cat, b_cat)
    q = qkvs[:, :_HC]
    bd = _block_diag_we(p["W_e"])
    qe = _mm(q, bd, None)

    disc = dis.reshape(_N, 1)
    kvx = jnp.concatenate(
        [qkvs[:, _HC:2 * _HC], qkvs[:, 2 * _HC:3 * _HC], x, disc,
         jnp.zeros((_N, _KVX_W - 2 * _HC - _C - 1), jnp.float32)], axis=1)
    qqe = jnp.concatenate(
        [q, qe, disc,
         jnp.zeros((_N, _QQE_W - _HC - _H * _DE - 1), jnp.float32)], axis=1)

    kvx_g = jnp.take(kvx, row_s, axis=0)
    qqe_g = jnp.take(qqe, col_s, axis=0)

    z = _edge_csum(kvx_g, qqe_g, ea_s, ew_s, tril)
    zb = jnp.take(z, indptr, axis=0)
    seg = zb[1:] - zb[:-1]

    num_v = seg[:, :_HC]
    xw_s = seg[:, _HC:_HC + _C]
    t_seg = seg[:, _HC + _C:_HC + _C + _H * _DE]
    denom = seg[:, _HC + _C + _H * _DE:_HC + _C + _H * _DE + _H]

    t_part = _mm(t_seg, bd.T, None)
    denom_rep = jnp.repeat(denom, _C, axis=1)
    xt = (num_v + t_part) / (denom_rep + 1e-16)
    xw_lin = _mm(xw_s, p["W_wl"].T, p["b_wl"])
    conv_out = xt + qkvs[:, 3 * _HC:] + xw_lin
    h_pre = _mm(conv_out, t_w.T, t_b)
    return _bn(h_pre, bn_g, bn_b, relu=True)


def kernel(x, edge_attr, edge_index, edge_weight, batch_index, params):
    row, col = edge_index[0], edge_index[1]
    perm = jnp.argsort(col)
    row_s = row[perm]
    col_s = col[perm]
    ea_s = edge_attr[perm]
    ew_s = edge_weight[perm].reshape(_E, 1)
    indptr = jnp.searchsorted(col_s, jnp.arange(_N + 1, dtype=jnp.int32)
                              ).astype(jnp.int32)
    tril = jnp.tril(jnp.ones((_TE, _TE), jnp.float32), k=-1)

    deg = _deg(row).reshape(_N)
    dis = deg ** -0.5

    prep1 = (row_s, col_s, ea_s, ew_s, indptr, dis, tril,
             params["bn1_g"], params["bn1_b"],
             params["transf1_W"], params["transf1_b"])
    h = _conv_layer(x, params["conv1"], prep1)
    prep2 = (row_s, col_s, ea_s, ew_s, indptr, dis, tril,
             params["bn2_g"], params["bn2_b"],
             params["transf2_W"], params["transf2_b"])
    h = _conv_layer(h, params["conv2"], prep2)

    kkeep = int(math.ceil(0.5 * _N))
    return _pool(h, params["pool_w"], kkeep)
